# baseline (speedup 1.0000x reference)
import jax
import jax.numpy as jnp
from jax.experimental import pallas as pl

BM = 1024
BITS = 6


def _lsh_kernel(x_ref, w_ref, gates_ref, dec_ref):
    xb = x_ref[...].astype(jnp.bfloat16)
    wb = w_ref[...].astype(jnp.bfloat16)
    h = jax.lax.dot_general(
        xb, wb, (((1,), (0,)), ((), ())),
        preferred_element_type=jnp.float32,
    )
    powers = (1 << jnp.arange(BITS - 1, -1, -1, dtype=jnp.int32)).astype(
        jnp.float32)
    dec = jnp.sum(jnp.where(h > 0, powers[None, :], 0.0), axis=1)
    dec_ref[...] = dec
    gates_ref[...] = jnp.ones_like(dec)


def kernel(x, W):
    n, d = x.shape
    grid = (n // BM,)
    gates, dec = pl.pallas_call(
        _lsh_kernel,
        grid=grid,
        in_specs=[
            pl.BlockSpec((BM, d), lambda i: (i, 0)),
            pl.BlockSpec((d, BITS), lambda i: (0, 0)),
        ],
        out_specs=[
            pl.BlockSpec((BM,), lambda i: (i,)),
            pl.BlockSpec((BM,), lambda i: (i,)),
        ],
        out_shape=[
            jax.ShapeDtypeStruct((n,), jnp.float32),
            jax.ShapeDtypeStruct((n,), jnp.float32),
        ],
    )(x, W)
    return gates, dec
